# baseline (speedup 1.0000x reference)
import jax
import jax.numpy as jnp
from jax import lax
from jax.experimental import pallas as pl
from jax.experimental.pallas import tpu as pltpu


R = 2048
C = 128
LOG_R = 11
LOG_C = 7
T = LOG_R + LOG_C


def _lovasz_kernel(logit_ref, truth_ref, out_ref, key_ref, val_ref):
    lg = logit_ref[0]
    tr = truth_ref[0]
    key_ref[:] = lg * (2.0 * tr - 1.0) - 1.0
    val_ref[:] = tr

    r_i = lax.broadcasted_iota(jnp.int32, (R, C), 0)
    c_i = lax.broadcasted_iota(jnp.int32, (R, C), 1)
    idx = c_i * R + r_i

    def one_pass(k, j):
        key = key_ref[:]
        val = val_ref[:]
        bit_j = (idx >> j) & 1
        bit_k = (idx >> k) & 1
        low = bit_j == 0

        def row_partner():
            s = jnp.int32(1) << j
            kp = jnp.where(low, pltpu.roll(key, R - s, 0), pltpu.roll(key, s, 0))
            vp = jnp.where(low, pltpu.roll(val, R - s, 0), pltpu.roll(val, s, 0))
            return kp, vp

        def lane_partner():
            m = jnp.int32(1) << (jnp.maximum(j - LOG_R, 0))
            kp = jnp.where(low, pltpu.roll(key, C - m, 1), pltpu.roll(key, m, 1))
            vp = jnp.where(low, pltpu.roll(val, C - m, 1), pltpu.roll(val, m, 1))
            return kp, vp

        kp, vp = lax.cond(j < LOG_R, row_partner, lane_partner)
        take_min = bit_j == bit_k
        sel_self = (take_min & (key <= kp)) | (~take_min & (key >= kp))
        key_ref[:] = jnp.where(sel_self, key, kp)
        val_ref[:] = jnp.where(sel_self, val, vp)

    def stage(k, carry):
        def inner(a, carry2):
            one_pass(k, k - 1 - a)
            return carry2
        return lax.fori_loop(0, k, inner, carry)

    lax.fori_loop(1, T + 1, stage, jnp.int32(0))

    val = val_ref[:]
    key = key_ref[:]

    def col_cumsum_step(t, cc):
        m = jnp.int32(1) << t
        sh = pltpu.roll(cc, m, 0)
        return cc + jnp.where(r_i >= m, sh, 0.0)

    colcum = lax.fori_loop(0, LOG_R, col_cumsum_step, val)
    colsum = colcum[R - 1 : R, :]

    lane_inc = colsum
    for t in range(LOG_C):
        m = 1 << t
        lane_inc = lane_inc + jnp.concatenate(
            [jnp.zeros((1, m), jnp.float32), lane_inc[:, : C - m]], axis=1
        )
    coloff = lane_inc - colsum
    cum = colcum + coloff
    gt = jnp.sum(val)

    rank1 = (idx + 1).astype(jnp.float32)
    n = rank1 - cum
    err = -key
    relu = jnp.maximum(err, 0.0)
    den = gt + n
    pos_w = 1.0 / den
    d1 = den - 1.0
    neg_w = jnp.where(d1 > 0, (gt - cum) / jnp.where(d1 > 0, d1 * den, 1.0), 1.0)
    term = relu * (val * pos_w + (1.0 - val) * neg_w)
    out_ref[0] = jnp.sum(term, keepdims=True)


@jax.jit
def kernel(logit, truth):
    b = logit.shape[0]
    lg = logit.reshape(b, R, C)
    tr = truth.reshape(b, R, C).astype(jnp.float32)
    losses = pl.pallas_call(
        _lovasz_kernel,
        grid=(b,),
        in_specs=[
            pl.BlockSpec((1, R, C), lambda i: (i, 0, 0)),
            pl.BlockSpec((1, R, C), lambda i: (i, 0, 0)),
        ],
        out_specs=pl.BlockSpec((1, 1, 1), lambda i: (i, 0, 0)),
        out_shape=jax.ShapeDtypeStruct((b, 1, 1), jnp.float32),
        scratch_shapes=[
            pltpu.VMEM((R, C), jnp.float32),
            pltpu.VMEM((R, C), jnp.float32),
        ],
    )(lg, tr)
    return jnp.mean(losses)


# speedup vs baseline: 2.0712x; 2.0712x over previous
import jax
import jax.numpy as jnp
from jax import lax
from jax.experimental import pallas as pl
from jax.experimental.pallas import tpu as pltpu


R = 2048
C = 128
LOG_R = 11
LOG_C = 7
T = LOG_R + LOG_C


def _to_ordered_i32(x):
    b = lax.bitcast_convert_type(x, jnp.int32)
    return b ^ ((b >> 31) & jnp.int32(0x7FFFFFFF))


def _from_ordered_i32(k):
    b = k ^ ((k >> 31) & jnp.int32(0x7FFFFFFF))
    return lax.bitcast_convert_type(b, jnp.float32)


def _lovasz_kernel(logit_ref, truth_ref, out_ref, key_ref):
    lg = logit_ref[0]
    tr = truth_ref[0]
    key_f = lg * (2.0 * tr - 1.0) - 1.0
    k_i = _to_ordered_i32(key_f)
    key_ref[:] = (k_i & jnp.int32(~1)) | tr.astype(jnp.int32)

    r_i = lax.broadcasted_iota(jnp.int32, (R, C), 0)
    c_i = lax.broadcasted_iota(jnp.int32, (R, C), 1)
    idx = c_i * R + r_i

    def one_pass(k, j):
        key = key_ref[:]
        bit_j = (idx >> j) & 1
        bit_k = (idx >> k) & 1
        low = bit_j == 0

        def row_partner():
            s = jnp.int32(1) << j
            return jnp.where(low, pltpu.roll(key, R - s, 0), pltpu.roll(key, s, 0))

        def lane_partner():
            m = jnp.int32(1) << (jnp.maximum(j - LOG_R, 0))
            return jnp.where(low, pltpu.roll(key, C - m, 1), pltpu.roll(key, m, 1))

        kp = lax.cond(j < LOG_R, row_partner, lane_partner)
        take_min = bit_j == bit_k
        sel_self = (take_min & (key <= kp)) | (~take_min & (key >= kp))
        key_ref[:] = jnp.where(sel_self, key, kp)

    def stage(k, carry):
        def inner(a, carry2):
            one_pass(k, k - 1 - a)
            return carry2
        return lax.fori_loop(0, k, inner, carry)

    lax.fori_loop(1, T + 1, stage, jnp.int32(0))

    pk = key_ref[:]
    val = (pk & 1).astype(jnp.float32)
    err = -_from_ordered_i32(pk & jnp.int32(~1))

    def col_cumsum_step(t, cc):
        m = jnp.int32(1) << t
        sh = pltpu.roll(cc, m, 0)
        return cc + jnp.where(r_i >= m, sh, 0.0)

    colcum = lax.fori_loop(0, LOG_R, col_cumsum_step, val)
    colsum = colcum[R - 1 : R, :]

    lane_inc = colsum
    for t in range(LOG_C):
        m = 1 << t
        lane_inc = lane_inc + jnp.concatenate(
            [jnp.zeros((1, m), jnp.float32), lane_inc[:, : C - m]], axis=1
        )
    coloff = lane_inc - colsum
    cum = colcum + coloff
    gt = jnp.sum(val)

    rank1 = (idx + 1).astype(jnp.float32)
    n = rank1 - cum
    relu = jnp.maximum(err, 0.0)
    den = gt + n
    pos_w = 1.0 / den
    d1 = den - 1.0
    neg_w = jnp.where(d1 > 0, (gt - cum) / jnp.where(d1 > 0, d1 * den, 1.0), 1.0)
    term = relu * (val * pos_w + (1.0 - val) * neg_w)
    out_ref[0] = jnp.sum(term, keepdims=True)


@jax.jit
def kernel(logit, truth):
    b = logit.shape[0]
    lg = logit.reshape(b, R, C)
    tr = truth.reshape(b, R, C).astype(jnp.float32)
    losses = pl.pallas_call(
        _lovasz_kernel,
        grid=(b,),
        in_specs=[
            pl.BlockSpec((1, R, C), lambda i: (i, 0, 0)),
            pl.BlockSpec((1, R, C), lambda i: (i, 0, 0)),
        ],
        out_specs=pl.BlockSpec((1, 1, 1), lambda i: (i, 0, 0)),
        out_shape=jax.ShapeDtypeStruct((b, 1, 1), jnp.float32),
        scratch_shapes=[
            pltpu.VMEM((R, C), jnp.int32),
        ],
    )(lg, tr)
    return jnp.mean(losses)


# speedup vs baseline: 2.2489x; 1.0858x over previous
import jax
import jax.numpy as jnp
from jax import lax
from jax.experimental import pallas as pl
from jax.experimental.pallas import tpu as pltpu


R = 2048
C = 128
LOG_R = 11
LOG_C = 7
T = LOG_R + LOG_C


def _to_ordered_i32(x):
    b = lax.bitcast_convert_type(x, jnp.int32)
    return b ^ ((b >> 31) & jnp.int32(0x7FFFFFFF))


def _from_ordered_i32(k):
    b = k ^ ((k >> 31) & jnp.int32(0x7FFFFFFF))
    return lax.bitcast_convert_type(b, jnp.float32)


def _lovasz_kernel(logit_ref, truth_ref, out_ref, key_ref):
    lg = logit_ref[0]
    tr = truth_ref[0]
    key_f = lg * (2.0 * tr - 1.0) - 1.0
    k_i = _to_ordered_i32(key_f)
    key_ref[:] = (k_i & jnp.int32(~1)) | tr.astype(jnp.int32)

    r_i = lax.broadcasted_iota(jnp.int32, (R, C), 0)
    c_i = lax.broadcasted_iota(jnp.int32, (R, C), 1)
    idx = c_i * R + r_i

    def one_pass(k, j, axis, s):
        key = key_ref[:]
        span = R if axis == 0 else C
        y = pltpu.roll(key, span - s, axis)
        mn = jnp.minimum(key, y)
        mx = jnp.maximum(key, y)
        a = idx >> j
        tm = ((a ^ (idx >> k)) & 1) == 0
        low = (a & 1) == 0
        lowres = jnp.where(tm, mn, mx)
        highres = jnp.where(tm, mx, mn)
        key_ref[:] = jnp.where(low, lowres, pltpu.roll(highres, s, axis))

    def row_pass(k, j):
        one_pass(k, j, 0, jnp.int32(1) << j)

    def lane_pass(k, j):
        one_pass(k, j, 1, jnp.int32(1) << (j - LOG_R))

    def stage_a(k, carry):
        def body(a2, c2):
            row_pass(k, k - 1 - a2)
            return c2
        return lax.fori_loop(0, k, body, carry)

    lax.fori_loop(1, LOG_R + 1, stage_a, jnp.int32(0))

    def stage_b(k, carry):
        def lane_body(a2, c2):
            lane_pass(k, k - 1 - a2)
            return c2
        lax.fori_loop(0, k - LOG_R, lane_body, jnp.int32(0))

        def row_body(a2, c2):
            row_pass(k, LOG_R - 1 - a2)
            return c2
        return lax.fori_loop(0, LOG_R, row_body, carry)

    lax.fori_loop(LOG_R + 1, T + 1, stage_b, jnp.int32(0))

    pk = key_ref[:]
    val = (pk & 1).astype(jnp.float32)
    err = -_from_ordered_i32(pk & jnp.int32(~1))

    def col_cumsum_step(t, cc):
        m = jnp.int32(1) << t
        sh = pltpu.roll(cc, m, 0)
        return cc + jnp.where(r_i >= m, sh, 0.0)

    colcum = lax.fori_loop(0, LOG_R, col_cumsum_step, val)
    colsum = colcum[R - 1 : R, :]

    lane_inc = colsum
    for t in range(LOG_C):
        m = 1 << t
        lane_inc = lane_inc + jnp.concatenate(
            [jnp.zeros((1, m), jnp.float32), lane_inc[:, : C - m]], axis=1
        )
    coloff = lane_inc - colsum
    cum = colcum + coloff
    gt = jnp.sum(val)

    rank1 = (idx + 1).astype(jnp.float32)
    n = rank1 - cum
    relu = jnp.maximum(err, 0.0)
    den = gt + n
    pos_w = 1.0 / den
    d1 = den - 1.0
    neg_w = jnp.where(d1 > 0, (gt - cum) / jnp.where(d1 > 0, d1 * den, 1.0), 1.0)
    term = relu * (val * pos_w + (1.0 - val) * neg_w)
    out_ref[0] = jnp.sum(term, keepdims=True)


@jax.jit
def kernel(logit, truth):
    b = logit.shape[0]
    lg = logit.reshape(b, R, C)
    tr = truth.reshape(b, R, C).astype(jnp.float32)
    losses = pl.pallas_call(
        _lovasz_kernel,
        grid=(b,),
        in_specs=[
            pl.BlockSpec((1, R, C), lambda i: (i, 0, 0)),
            pl.BlockSpec((1, R, C), lambda i: (i, 0, 0)),
        ],
        out_specs=pl.BlockSpec((1, 1, 1), lambda i: (i, 0, 0)),
        out_shape=jax.ShapeDtypeStruct((b, 1, 1), jnp.float32),
        scratch_shapes=[
            pltpu.VMEM((R, C), jnp.int32),
        ],
    )(lg, tr)
    return jnp.mean(losses)


# speedup vs baseline: 7.8796x; 3.5037x over previous
import jax
import jax.numpy as jnp
from jax import lax
from jax.experimental import pallas as pl
from jax.experimental.pallas import tpu as pltpu


R = 2048
C = 128
LOG_R = 11
LOG_C = 7
T = LOG_R + LOG_C


def _to_ordered_i32(x):
    b = lax.bitcast_convert_type(x, jnp.int32)
    return b ^ ((b >> 31) & jnp.int32(0x7FFFFFFF))


def _from_ordered_i32(k):
    b = k ^ ((k >> 31) & jnp.int32(0x7FFFFFFF))
    return lax.bitcast_convert_type(b, jnp.float32)


PAD = 1024


def _lovasz_kernel(logit_ref, truth_ref, out_ref, key_ref):
    lg = logit_ref[0]
    tr = truth_ref[0]
    key_f = lg * (2.0 * tr - 1.0) - 1.0
    k_i = _to_ordered_i32(key_f)
    key_ref[PAD : PAD + R, :] = (k_i & jnp.int32(~1)) | tr.astype(jnp.int32)

    r_i = lax.broadcasted_iota(jnp.int32, (R, C), 0)
    c_i = lax.broadcasted_iota(jnp.int32, (R, C), 1)
    idx = (c_i << LOG_R) | r_i

    def row_pass(k, j):
        s = jnp.int32(1) << j
        key = key_ref[PAD : PAD + R, :]
        yp = key_ref[pl.ds(PAD + s, R), :]
        ym = key_ref[pl.ds(PAD - s, R), :]
        aj = (idx >> j) & 1
        bk = (idx >> k) & 1
        kp = jnp.where(aj == 0, yp, ym)
        mn = jnp.minimum(key, kp)
        mx = jnp.maximum(key, kp)
        key_ref[PAD : PAD + R, :] = jnp.where(aj == bk, mn, mx)

    def lane_pass(k, j):
        m = jnp.int32(1) << (j - LOG_R)
        key = key_ref[PAD : PAD + R, :]
        yp = pltpu.roll(key, C - m, 1)
        ym = pltpu.roll(key, m, 1)
        aj = (idx >> j) & 1
        bk = (idx >> k) & 1
        kp = jnp.where(aj == 0, yp, ym)
        mn = jnp.minimum(key, kp)
        mx = jnp.maximum(key, kp)
        key_ref[PAD : PAD + R, :] = jnp.where(aj == bk, mn, mx)

    def stage_a(k, carry):
        def body(a2, c2):
            row_pass(k, k - 1 - a2)
            return c2
        return lax.fori_loop(0, k, body, carry)

    lax.fori_loop(1, LOG_R + 1, stage_a, jnp.int32(0))

    def stage_b(k, carry):
        def lane_body(a2, c2):
            lane_pass(k, k - 1 - a2)
            return c2
        lax.fori_loop(0, k - LOG_R, lane_body, jnp.int32(0))

        def row_body(a2, c2):
            row_pass(k, LOG_R - 1 - a2)
            return c2
        return lax.fori_loop(0, LOG_R, row_body, carry)

    lax.fori_loop(LOG_R + 1, T + 1, stage_b, jnp.int32(0))

    pk = key_ref[PAD : PAD + R, :]
    val = (pk & 1).astype(jnp.float32)
    err = -_from_ordered_i32(pk & jnp.int32(~1))

    def col_cumsum_step(t, cc):
        m = jnp.int32(1) << t
        sh = pltpu.roll(cc, m, 0)
        return cc + jnp.where(r_i >= m, sh, 0.0)

    colcum = lax.fori_loop(0, LOG_R, col_cumsum_step, val)
    colsum = colcum[R - 1 : R, :]

    lane_inc = colsum
    for t in range(LOG_C):
        m = 1 << t
        lane_inc = lane_inc + jnp.concatenate(
            [jnp.zeros((1, m), jnp.float32), lane_inc[:, : C - m]], axis=1
        )
    coloff = lane_inc - colsum
    cum = colcum + coloff
    gt = jnp.sum(val)

    rank1 = (idx + 1).astype(jnp.float32)
    n = rank1 - cum
    relu = jnp.maximum(err, 0.0)
    den = gt + n
    pos_w = 1.0 / den
    d1 = den - 1.0
    neg_w = jnp.where(d1 > 0, (gt - cum) / jnp.where(d1 > 0, d1 * den, 1.0), 1.0)
    term = relu * (val * pos_w + (1.0 - val) * neg_w)
    out_ref[0] = jnp.sum(term, keepdims=True)


@jax.jit
def kernel(logit, truth):
    b = logit.shape[0]
    lg = logit.reshape(b, R, C)
    tr = truth.reshape(b, R, C).astype(jnp.float32)
    losses = pl.pallas_call(
        _lovasz_kernel,
        grid=(b,),
        in_specs=[
            pl.BlockSpec((1, R, C), lambda i: (i, 0, 0)),
            pl.BlockSpec((1, R, C), lambda i: (i, 0, 0)),
        ],
        out_specs=pl.BlockSpec((1, 1, 1), lambda i: (i, 0, 0)),
        out_shape=jax.ShapeDtypeStruct((b, 1, 1), jnp.float32),
        scratch_shapes=[
            pltpu.VMEM((R + 2 * PAD, C), jnp.int32),
        ],
    )(lg, tr)
    return jnp.mean(losses)


# speedup vs baseline: 9.3623x; 1.1882x over previous
import jax
import jax.numpy as jnp
from jax import lax
from jax.experimental import pallas as pl
from jax.experimental.pallas import tpu as pltpu


R = 2048
C = 128
LOG_R = 11
LOG_C = 7
T = LOG_R + LOG_C


def _to_ordered_i32(x):
    b = lax.bitcast_convert_type(x, jnp.int32)
    return b ^ ((b >> 31) & jnp.int32(0x7FFFFFFF))


def _from_ordered_i32(k):
    b = k ^ ((k >> 31) & jnp.int32(0x7FFFFFFF))
    return lax.bitcast_convert_type(b, jnp.float32)


PAD = 1024


def _lovasz_kernel(logit_ref, truth_ref, out_ref, key_ref, cc_ref):
    lg = logit_ref[0]
    tr = truth_ref[0]
    key_f = lg * (2.0 * tr - 1.0) - 1.0
    k_i = _to_ordered_i32(key_f)
    key_ref[PAD : PAD + R, :] = (k_i & jnp.int32(~1)) | tr.astype(jnp.int32)

    r_i = lax.broadcasted_iota(jnp.int32, (R, C), 0)
    c_i = lax.broadcasted_iota(jnp.int32, (R, C), 1)

    def row_pass(j, bk):
        s = jnp.int32(1) << j
        key = key_ref[PAD : PAD + R, :]
        yp = key_ref[pl.ds(PAD + s, R), :]
        ym = key_ref[pl.ds(PAD - s, R), :]
        aj = (r_i >> j) & 1
        kp = jnp.where(aj == 0, yp, ym)
        mn = jnp.minimum(key, kp)
        mx = jnp.maximum(key, kp)
        key_ref[PAD : PAD + R, :] = jnp.where(aj == bk, mn, mx)

    def lane_pass(jj, bk):
        m = jnp.int32(1) << jj
        key = key_ref[PAD : PAD + R, :]
        yp = pltpu.roll(key, C - m, 1)
        ym = pltpu.roll(key, m, 1)
        aj = (c_i >> jj) & 1
        kp = jnp.where(aj == 0, yp, ym)
        mn = jnp.minimum(key, kp)
        mx = jnp.maximum(key, kp)
        key_ref[PAD : PAD + R, :] = jnp.where(aj == bk, mn, mx)

    def stage_a(k, carry):
        def body(a2, c2):
            j = k - 1 - a2
            row_pass(j, (r_i >> k) & 1)
            return c2
        return lax.fori_loop(0, k, body, carry)

    lax.fori_loop(1, LOG_R, stage_a, jnp.int32(0))

    def stage11_body(a2, c2):
        row_pass(LOG_R - 1 - a2, c_i & 1)
        return c2

    lax.fori_loop(0, LOG_R, stage11_body, jnp.int32(0))

    def stage_b(k, carry):
        kk = k - LOG_R

        def lane_body(a2, c2):
            lane_pass(kk - 1 - a2, (c_i >> kk) & 1)
            return c2
        lax.fori_loop(0, kk, lane_body, jnp.int32(0))

        def row_body(a2, c2):
            row_pass(LOG_R - 1 - a2, (c_i >> kk) & 1)
            return c2
        return lax.fori_loop(0, LOG_R, row_body, carry)

    lax.fori_loop(LOG_R + 1, T + 1, stage_b, jnp.int32(0))

    pk = key_ref[PAD : PAD + R, :]
    val = (pk & 1).astype(jnp.float32)
    err = -_from_ordered_i32(pk & jnp.int32(~1))

    cc_ref[PAD : PAD + R, :] = val

    def col_cumsum_step(t, carry):
        m = jnp.int32(1) << t
        cc = cc_ref[PAD : PAD + R, :]
        sh = cc_ref[pl.ds(PAD - m, R), :]
        cc_ref[PAD : PAD + R, :] = cc + jnp.where(r_i >= m, sh, 0.0)
        return carry

    lax.fori_loop(0, LOG_R, col_cumsum_step, jnp.int32(0))
    colcum = cc_ref[PAD : PAD + R, :]
    colsum = colcum[R - 1 : R, :]

    lane_inc = colsum
    for t in range(LOG_C):
        m = 1 << t
        lane_inc = lane_inc + jnp.concatenate(
            [jnp.zeros((1, m), jnp.float32), lane_inc[:, : C - m]], axis=1
        )
    coloff = lane_inc - colsum
    cum = colcum + coloff
    gt = jnp.sum(val)

    rank1 = (((c_i << LOG_R) | r_i) + 1).astype(jnp.float32)
    n = rank1 - cum
    relu = jnp.maximum(err, 0.0)
    den = gt + n
    pos_w = 1.0 / den
    d1 = den - 1.0
    neg_w = jnp.where(d1 > 0, (gt - cum) / jnp.where(d1 > 0, d1 * den, 1.0), 1.0)
    term = relu * (val * pos_w + (1.0 - val) * neg_w)
    out_ref[0] = jnp.sum(term, keepdims=True)


@jax.jit
def kernel(logit, truth):
    b = logit.shape[0]
    lg = logit.reshape(b, R, C)
    tr = truth.reshape(b, R, C).astype(jnp.float32)
    losses = pl.pallas_call(
        _lovasz_kernel,
        grid=(b,),
        in_specs=[
            pl.BlockSpec((1, R, C), lambda i: (i, 0, 0)),
            pl.BlockSpec((1, R, C), lambda i: (i, 0, 0)),
        ],
        out_specs=pl.BlockSpec((1, 1, 1), lambda i: (i, 0, 0)),
        out_shape=jax.ShapeDtypeStruct((b, 1, 1), jnp.float32),
        scratch_shapes=[
            pltpu.VMEM((R + 2 * PAD, C), jnp.int32),
            pltpu.VMEM((R + PAD, C), jnp.float32),
        ],
    )(lg, tr)
    return jnp.mean(losses)


# speedup vs baseline: 9.5376x; 1.0187x over previous
import jax
import jax.numpy as jnp
from jax import lax
from jax.experimental import pallas as pl
from jax.experimental.pallas import tpu as pltpu


R = 2048
C = 128
LOG_R = 11
LOG_C = 7
T = LOG_R + LOG_C


def _to_ordered_i32(x):
    b = lax.bitcast_convert_type(x, jnp.int32)
    return b ^ ((b >> 31) & jnp.int32(0x7FFFFFFF))


def _from_ordered_i32(k):
    b = k ^ ((k >> 31) & jnp.int32(0x7FFFFFFF))
    return lax.bitcast_convert_type(b, jnp.float32)


PAD = 1024


def _lovasz_kernel(logit_ref, truth_ref, out_ref, key_ref, cc_ref):
    lg = logit_ref[0]
    tr = truth_ref[0]
    key_f = lg * (2.0 * tr - 1.0) - 1.0
    k_i = _to_ordered_i32(key_f)
    key_ref[PAD : PAD + R, :] = (k_i & jnp.int32(~1)) | tr.astype(jnp.int32)

    r_i = lax.broadcasted_iota(jnp.int32, (R, C), 0)
    c_i = lax.broadcasted_iota(jnp.int32, (R, C), 1)

    def row_pass(j, bk):
        s = jnp.int32(1) << j
        key = key_ref[PAD : PAD + R, :]
        yp = key_ref[pl.ds(PAD + s, R), :]
        ym = key_ref[pl.ds(PAD - s, R), :]
        aj = (r_i >> j) & 1
        kp = jnp.where(aj == 0, yp, ym)
        mn = jnp.minimum(key, kp)
        mx = jnp.maximum(key, kp)
        key_ref[PAD : PAD + R, :] = jnp.where(aj == bk, mn, mx)

    def lane_pass(jj, kk):
        m = 1 << jj
        key = key_ref[PAD : PAD + R, :]
        yp = jnp.concatenate([key[:, m:], key[:, :m]], axis=1)
        ym = jnp.concatenate([key[:, C - m :], key[:, : C - m]], axis=1)
        aj = (c_i >> jj) & 1
        bk = (c_i >> kk) & 1
        kp = jnp.where(aj == 0, yp, ym)
        mn = jnp.minimum(key, kp)
        mx = jnp.maximum(key, kp)
        key_ref[PAD : PAD + R, :] = jnp.where(aj == bk, mn, mx)

    def stage_a(k, carry):
        def body(a2, c2):
            j = k - 1 - a2
            row_pass(j, (r_i >> k) & 1)
            return c2
        return lax.fori_loop(0, k, body, carry)

    lax.fori_loop(1, LOG_R, stage_a, jnp.int32(0))

    def stage11_body(a2, c2):
        row_pass(LOG_R - 1 - a2, c_i & 1)
        return c2

    lax.fori_loop(0, LOG_R, stage11_body, jnp.int32(0))

    for k in range(LOG_R + 1, T + 1):
        kk = k - LOG_R
        for jj in range(kk - 1, -1, -1):
            lane_pass(jj, kk)

        def row_body(a2, c2, _kk=kk):
            row_pass(LOG_R - 1 - a2, (c_i >> _kk) & 1)
            return c2
        lax.fori_loop(0, LOG_R, row_body, jnp.int32(0))

    pk = key_ref[PAD : PAD + R, :]
    val = (pk & 1).astype(jnp.float32)
    err = -_from_ordered_i32(pk & jnp.int32(~1))

    cc_ref[PAD : PAD + R, :] = val

    def col_cumsum_step(t, carry):
        m = jnp.int32(1) << t
        cc = cc_ref[PAD : PAD + R, :]
        sh = cc_ref[pl.ds(PAD - m, R), :]
        cc_ref[PAD : PAD + R, :] = cc + jnp.where(r_i >= m, sh, 0.0)
        return carry

    lax.fori_loop(0, LOG_R, col_cumsum_step, jnp.int32(0))
    colcum = cc_ref[PAD : PAD + R, :]
    colsum = colcum[R - 1 : R, :]

    lane_inc = colsum
    for t in range(LOG_C):
        m = 1 << t
        lane_inc = lane_inc + jnp.concatenate(
            [jnp.zeros((1, m), jnp.float32), lane_inc[:, : C - m]], axis=1
        )
    coloff = lane_inc - colsum
    cum = colcum + coloff
    gt = jnp.sum(val)

    rank1 = (((c_i << LOG_R) | r_i) + 1).astype(jnp.float32)
    n = rank1 - cum
    relu = jnp.maximum(err, 0.0)
    den = gt + n
    pos_w = 1.0 / den
    d1 = den - 1.0
    neg_w = jnp.where(d1 > 0, (gt - cum) / jnp.where(d1 > 0, d1 * den, 1.0), 1.0)
    term = relu * (val * pos_w + (1.0 - val) * neg_w)
    out_ref[0] = jnp.sum(term, keepdims=True)


@jax.jit
def kernel(logit, truth):
    b = logit.shape[0]
    lg = logit.reshape(b, R, C)
    tr = truth.reshape(b, R, C).astype(jnp.float32)
    losses = pl.pallas_call(
        _lovasz_kernel,
        grid=(b,),
        in_specs=[
            pl.BlockSpec((1, R, C), lambda i: (i, 0, 0)),
            pl.BlockSpec((1, R, C), lambda i: (i, 0, 0)),
        ],
        out_specs=pl.BlockSpec((1, 1, 1), lambda i: (i, 0, 0)),
        out_shape=jax.ShapeDtypeStruct((b, 1, 1), jnp.float32),
        scratch_shapes=[
            pltpu.VMEM((R + 2 * PAD, C), jnp.int32),
            pltpu.VMEM((R + PAD, C), jnp.float32),
        ],
    )(lg, tr)
    return jnp.mean(losses)
